# Initial kernel scaffold; baseline (speedup 1.0000x reference)
import jax
import jax.numpy as jnp
from jax.experimental import pallas as pl


def kernel(router_logits):
    raise NotImplementedError("write your pallas kernel here")


# speedup vs baseline: 1.3327x; 1.3327x over previous
import functools

import jax
import jax.numpy as jnp
from jax import lax
from jax.experimental import pallas as pl
from jax.experimental.pallas import tpu as pltpu
from jax.experimental.pallas import tpu_sc as plsc

NUM_CORES = 2
NUM_SUBCORES = 16
LANES = 16
NUM_WORKERS = NUM_CORES * NUM_SUBCORES

ROWS = 32768
EXPERTS = 64
ROWS_PER_W = ROWS // NUM_WORKERS
CHUNK_ROWS = 256
NUM_CHUNKS = ROWS_PER_W // CHUNK_ROWS
GROUPS = CHUNK_ROWS // LANES

_mesh = plsc.VectorSubcoreMesh(
    core_axis_name="c",
    subcore_axis_name="s",
    num_cores=NUM_CORES,
    num_subcores=NUM_SUBCORES,
)


@functools.partial(
    pl.kernel,
    out_type=(
        jax.ShapeDtypeStruct((ROWS, 2), jnp.float32),
        jax.ShapeDtypeStruct((ROWS, 2), jnp.int32),
    ),
    mesh=_mesh,
    scratch_types=(
        pltpu.VMEM((CHUNK_ROWS * EXPERTS,), jnp.float32),
        pltpu.VMEM((CHUNK_ROWS, 2), jnp.float32),
        pltpu.VMEM((CHUNK_ROWS, 2), jnp.int32),
    ),
    compiler_params=pltpu.CompilerParams(needs_layout_passes=False),
)
def _router(logits_hbm, w_hbm, ids_hbm, in_v, w_v, ids_v):
    wid = lax.axis_index("s") * NUM_CORES + lax.axis_index("c")
    base = wid * ROWS_PER_W

    lane = lax.iota(jnp.int32, LANES)
    zero = jnp.zeros((LANES,), jnp.int32)
    one = jnp.ones((LANES,), jnp.int32)

    def chunk_body(ch, carry):
        row0 = base + ch * CHUNK_ROWS
        pltpu.sync_copy(
            logits_hbm.at[pl.ds(row0 * EXPERTS, CHUNK_ROWS * EXPERTS)], in_v
        )

        def group_body(g, carry_in):
            row_idx = g * LANES + lane
            flat0 = row_idx * EXPERTS
            m1 = plsc.load_gather(in_v, [flat0])
            i1 = zero
            m2 = jnp.full((LANES,), -jnp.inf, jnp.float32)
            i2 = zero
            for e in range(1, EXPERTS):
                v = plsc.load_gather(in_v, [flat0 + e])
                e_vec = jnp.full((LANES,), e, jnp.int32)
                gt1 = v > m1
                gt2 = v > m2
                m2 = jnp.where(gt1, m1, jnp.where(gt2, v, m2))
                i2 = jnp.where(gt1, i1, jnp.where(gt2, e_vec, i2))
                m1 = jnp.where(gt1, v, m1)
                i1 = jnp.where(gt1, e_vec, i1)
            ex = jnp.exp(m2 - m1)
            s = 1.0 + ex
            w1 = 1.0 / s
            w2 = ex / s
            plsc.store_scatter(w_v, [row_idx, zero], w1)
            plsc.store_scatter(w_v, [row_idx, one], w2)
            plsc.store_scatter(ids_v, [row_idx, zero], i1)
            plsc.store_scatter(ids_v, [row_idx, one], i2)
            return carry_in

        lax.fori_loop(0, GROUPS, group_body, 0)
        pltpu.sync_copy(w_v, w_hbm.at[pl.ds(row0, CHUNK_ROWS)])
        pltpu.sync_copy(ids_v, ids_hbm.at[pl.ds(row0, CHUNK_ROWS)])
        return carry

    lax.fori_loop(0, NUM_CHUNKS, chunk_body, 0)


def kernel(router_logits):
    flat = router_logits.reshape(-1)
    topk_weights, topk_ids = _router(flat)
    return (topk_weights, topk_ids, router_logits)
